# baseline (speedup 1.0000x reference)
import jax
import jax.numpy as jnp
from jax.experimental import pallas as pl
from jax.experimental.pallas import tpu as pltpu


def _ffn_body(x_ref, af_ref, w1_ref, b1_ref, w2_ref, b2_ref,
              wr1x_ref, wr1a_ref, br1_ref, wr2_ref, br2_ref, out_ref):
    k = pl.program_id(1)
    nk = pl.num_programs(1)

    x = x_ref[...]
    h = jnp.dot(x, w1_ref[...], preferred_element_type=jnp.float32)
    h = h + b1_ref[...]
    c0 = jnp.float32(0.7978845608028654)
    c1 = jnp.float32(0.044715)
    inner = c0 * (h + c1 * h * h * h)
    h = jnp.float32(0.5) * h * (jnp.float32(1.0) + jnp.tanh(inner))
    part = jnp.dot(h.astype(x.dtype), w2_ref[...],
                   preferred_element_type=jnp.float32)

    @pl.when(k == 0)
    def _init():
        out_ref[...] = jnp.zeros_like(out_ref)

    out_ref[...] += part

    @pl.when(k == nk - 1)
    def _finalize():
        g = jnp.dot(x, wr1x_ref[...], preferred_element_type=jnp.float32)
        g = g + af_ref[...] * wr1a_ref[...]
        g = g + br1_ref[...]
        g = jnp.float32(0.5) * g * (
            jnp.float32(1.0) + jax.lax.erf(g * jnp.float32(0.7071067811865476)))
        logits = jnp.dot(g, wr2_ref[...],
                         preferred_element_type=jnp.float32) + br2_ref[...]
        probs = jax.nn.sigmoid(logits)
        out_ref[...] = (out_ref[...] + b2_ref[...]) * probs


def kernel(x, attn_feat, w1, b1, w2, b2, wr1, br1, wr2, br2):
    B, S, H = x.shape
    N = B * S
    DFF = w1.shape[1]
    GH = wr1.shape[1]
    cdtype = jnp.bfloat16

    xr = x.reshape(N, H).astype(cdtype)
    af = attn_feat.reshape(N, 1)
    wr1x = wr1[:H].astype(cdtype)
    wr1a = wr1[H:H + 1]
    b1r = b1.reshape(1, DFF)
    b2r = b2.reshape(1, H)
    br1r = br1.reshape(1, GH)
    br2r = br2.reshape(1, 1)

    bm = min(512, N)
    bk = min(1024, DFF)
    grid = (N // bm, DFF // bk)

    out = pl.pallas_call(
        _ffn_body,
        grid=grid,
        in_specs=[
            pl.BlockSpec((bm, H), lambda i, k: (i, 0)),
            pl.BlockSpec((bm, 1), lambda i, k: (i, 0)),
            pl.BlockSpec((H, bk), lambda i, k: (0, k)),
            pl.BlockSpec((1, bk), lambda i, k: (0, k)),
            pl.BlockSpec((bk, H), lambda i, k: (k, 0)),
            pl.BlockSpec((1, H), lambda i, k: (0, 0)),
            pl.BlockSpec((H, GH), lambda i, k: (0, 0)),
            pl.BlockSpec((1, GH), lambda i, k: (0, 0)),
            pl.BlockSpec((1, GH), lambda i, k: (0, 0)),
            pl.BlockSpec((GH, 1), lambda i, k: (0, 0)),
            pl.BlockSpec((1, 1), lambda i, k: (0, 0)),
        ],
        out_specs=pl.BlockSpec((bm, H), lambda i, k: (i, 0)),
        out_shape=jax.ShapeDtypeStruct((N, H), jnp.float32),
        compiler_params=pltpu.CompilerParams(
            dimension_semantics=("parallel", "arbitrary")),
    )(xr, af, w1.astype(cdtype), b1r, w2.astype(cdtype), b2r,
      wr1x, wr1a, br1r, wr2, br2r)

    return out.reshape(B, S, H)


# speedup vs baseline: 1.0735x; 1.0735x over previous
import jax
import jax.numpy as jnp
from jax.experimental import pallas as pl
from jax.experimental.pallas import tpu as pltpu


def _ffn_body(x_ref, af_ref, w1_ref, b1_ref, w2_ref, b2_ref,
              wr1x_ref, wr1a_ref, br1_ref, wr2_ref, br2_ref, out_ref):
    k = pl.program_id(1)
    nk = pl.num_programs(1)

    x = x_ref[...]
    h = jnp.dot(x, w1_ref[...], preferred_element_type=jnp.float32)
    h = h + b1_ref[...]
    c0 = jnp.float32(0.7978845608028654)
    c1 = jnp.float32(0.044715)
    inner = c0 * (h + c1 * h * h * h)
    h = jnp.float32(0.5) * h * (jnp.float32(1.0) + jnp.tanh(inner))
    part = jnp.dot(h.astype(x.dtype), w2_ref[...],
                   preferred_element_type=jnp.float32)

    @pl.when(k == 0)
    def _init():
        out_ref[...] = jnp.zeros_like(out_ref)

    out_ref[...] += part

    @pl.when(k == nk - 1)
    def _finalize():
        g = jnp.dot(x, wr1x_ref[...], preferred_element_type=jnp.float32)
        g = g + af_ref[...] * wr1a_ref[...]
        g = g + br1_ref[...]
        g = jnp.float32(0.5) * g * (
            jnp.float32(1.0) + jax.lax.erf(g * jnp.float32(0.7071067811865476)))
        logits = jnp.dot(g, wr2_ref[...],
                         preferred_element_type=jnp.float32) + br2_ref[...]
        probs = jax.nn.sigmoid(logits)
        out_ref[...] = (out_ref[...] + b2_ref[...]) * probs


def kernel(x, attn_feat, w1, b1, w2, b2, wr1, br1, wr2, br2):
    B, S, H = x.shape
    N = B * S
    DFF = w1.shape[1]
    GH = wr1.shape[1]
    cdtype = jnp.bfloat16

    xr = x.reshape(N, H).astype(cdtype)
    af = attn_feat.reshape(N, 1)
    wr1x = wr1[:H].astype(cdtype)
    wr1a = wr1[H:H + 1]
    b1r = b1.reshape(1, DFF)
    b2r = b2.reshape(1, H)
    br1r = br1.reshape(1, GH)
    br2r = br2.reshape(1, 1)

    bm = min(1024, N)
    bk = min(1024, DFF)
    grid = (N // bm, DFF // bk)

    out = pl.pallas_call(
        _ffn_body,
        grid=grid,
        in_specs=[
            pl.BlockSpec((bm, H), lambda i, k: (i, 0)),
            pl.BlockSpec((bm, 1), lambda i, k: (i, 0)),
            pl.BlockSpec((H, bk), lambda i, k: (0, k)),
            pl.BlockSpec((1, bk), lambda i, k: (0, k)),
            pl.BlockSpec((bk, H), lambda i, k: (k, 0)),
            pl.BlockSpec((1, H), lambda i, k: (0, 0)),
            pl.BlockSpec((H, GH), lambda i, k: (0, 0)),
            pl.BlockSpec((1, GH), lambda i, k: (0, 0)),
            pl.BlockSpec((1, GH), lambda i, k: (0, 0)),
            pl.BlockSpec((GH, 1), lambda i, k: (0, 0)),
            pl.BlockSpec((1, 1), lambda i, k: (0, 0)),
        ],
        out_specs=pl.BlockSpec((bm, H), lambda i, k: (i, 0)),
        out_shape=jax.ShapeDtypeStruct((N, H), jnp.float32),
        compiler_params=pltpu.CompilerParams(
            dimension_semantics=("parallel", "arbitrary")),
    )(xr, af, w1.astype(cdtype), b1r, w2.astype(cdtype), b2r,
      wr1x, wr1a, br1r, wr2, br2r)

    return out.reshape(B, S, H)


# speedup vs baseline: 1.1217x; 1.0449x over previous
import jax
import jax.numpy as jnp
from jax.experimental import pallas as pl
from jax.experimental.pallas import tpu as pltpu


def _ffn_body(x_ref, af_ref, w1_ref, b1_ref, w2_ref, b2_ref,
              wr1x_ref, wr1a_ref, br1_ref, wr2_ref, br2_ref, out_ref):
    k = pl.program_id(1)
    nk = pl.num_programs(1)

    x = x_ref[...]
    bk = w1_ref.shape[1]
    nsub = 2
    sk = bk // nsub
    c0 = jnp.float32(0.7978845608028654)
    c1 = jnp.float32(0.044715)

    parts = []
    for s in range(nsub):
        sl = pl.ds(s * sk, sk)
        h = jnp.dot(x, w1_ref[:, sl], preferred_element_type=jnp.float32)
        h = h + b1_ref[:, sl]
        inner = c0 * (h + c1 * h * h * h)
        h = jnp.float32(0.5) * h * (jnp.float32(1.0) + jnp.tanh(inner))
        parts.append(jnp.dot(h.astype(x.dtype), w2_ref[sl, :],
                             preferred_element_type=jnp.float32))

    part = parts[0] + parts[1]

    @pl.when(k == 0)
    def _init():
        out_ref[...] = jnp.zeros_like(out_ref)

    out_ref[...] += part

    @pl.when(k == nk - 1)
    def _finalize():
        g = jnp.dot(x, wr1x_ref[...], preferred_element_type=jnp.float32)
        g = g + af_ref[...] * wr1a_ref[...]
        g = g + br1_ref[...]
        g = jnp.float32(0.5) * g * (
            jnp.float32(1.0) + jax.lax.erf(g * jnp.float32(0.7071067811865476)))
        logits = jnp.dot(g, wr2_ref[...],
                         preferred_element_type=jnp.float32) + br2_ref[...]
        probs = jax.nn.sigmoid(logits)
        out_ref[...] = (out_ref[...] + b2_ref[...]) * probs


def kernel(x, attn_feat, w1, b1, w2, b2, wr1, br1, wr2, br2):
    B, S, H = x.shape
    N = B * S
    DFF = w1.shape[1]
    GH = wr1.shape[1]
    cdtype = jnp.bfloat16

    xr = x.reshape(N, H).astype(cdtype)
    af = attn_feat.reshape(N, 1)
    wr1x = wr1[:H].astype(cdtype)
    wr1a = wr1[H:H + 1]
    b1r = b1.reshape(1, DFF)
    b2r = b2.reshape(1, H)
    br1r = br1.reshape(1, GH)
    br2r = br2.reshape(1, 1)

    bm = min(512, N)
    bk = min(2048, DFF)
    grid = (N // bm, DFF // bk)

    out = pl.pallas_call(
        _ffn_body,
        grid=grid,
        in_specs=[
            pl.BlockSpec((bm, H), lambda i, k: (i, 0)),
            pl.BlockSpec((bm, 1), lambda i, k: (i, 0)),
            pl.BlockSpec((H, bk), lambda i, k: (0, k)),
            pl.BlockSpec((1, bk), lambda i, k: (0, k)),
            pl.BlockSpec((bk, H), lambda i, k: (k, 0)),
            pl.BlockSpec((1, H), lambda i, k: (0, 0)),
            pl.BlockSpec((H, GH), lambda i, k: (0, 0)),
            pl.BlockSpec((1, GH), lambda i, k: (0, 0)),
            pl.BlockSpec((1, GH), lambda i, k: (0, 0)),
            pl.BlockSpec((GH, 1), lambda i, k: (0, 0)),
            pl.BlockSpec((1, 1), lambda i, k: (0, 0)),
        ],
        out_specs=pl.BlockSpec((bm, H), lambda i, k: (i, 0)),
        out_shape=jax.ShapeDtypeStruct((N, H), jnp.float32),
        compiler_params=pltpu.CompilerParams(
            dimension_semantics=("parallel", "arbitrary")),
    )(xr, af, w1.astype(cdtype), b1r, w2.astype(cdtype), b2r,
      wr1x, wr1a, br1r, wr2, br2r)

    return out.reshape(B, S, H)


# speedup vs baseline: 1.1305x; 1.0078x over previous
import jax
import jax.numpy as jnp
from jax.experimental import pallas as pl
from jax.experimental.pallas import tpu as pltpu


def _ffn_body(x_ref, af_ref, w1_ref, b1_ref, w2_ref, b2_ref,
              wr1x_ref, wr1a_ref, br1_ref, wr2_ref, br2_ref, out_ref):
    k = pl.program_id(1)
    nk = pl.num_programs(1)

    x = x_ref[...]
    bk = w1_ref.shape[1]
    nsub = 2
    sk = bk // nsub
    c0 = jnp.float32(0.7978845608028654)
    c1 = jnp.float32(0.044715)

    parts = []
    for s in range(nsub):
        sl = pl.ds(s * sk, sk)
        h = jnp.dot(x, w1_ref[:, sl], preferred_element_type=jnp.float32)
        h = h + b1_ref[:, sl]
        inner = c0 * (h + c1 * h * h * h)
        h = jnp.float32(0.5) * h * (jnp.float32(1.0) + jnp.tanh(inner))
        parts.append(jnp.dot(h.astype(x.dtype), w2_ref[sl, :],
                             preferred_element_type=jnp.float32))

    part = sum(parts[1:], parts[0])

    @pl.when(k == 0)
    def _init():
        out_ref[...] = part

    @pl.when(k != 0)
    def _accum():
        out_ref[...] += part

    @pl.when(k == nk - 1)
    def _finalize():
        g = jnp.dot(x, wr1x_ref[...], preferred_element_type=jnp.float32)
        g = g + af_ref[...] * wr1a_ref[...]
        g = g + br1_ref[...]
        g = jnp.float32(0.5) * g * (
            jnp.float32(1.0) + jax.lax.erf(g * jnp.float32(0.7071067811865476)))
        logits = jnp.dot(g, wr2_ref[...],
                         preferred_element_type=jnp.float32) + br2_ref[...]
        probs = jax.nn.sigmoid(logits)
        out_ref[...] = (out_ref[...] + b2_ref[...]) * probs


def kernel(x, attn_feat, w1, b1, w2, b2, wr1, br1, wr2, br2):
    B, S, H = x.shape
    N = B * S
    DFF = w1.shape[1]
    GH = wr1.shape[1]
    cdtype = jnp.bfloat16

    xr = x.reshape(N, H).astype(cdtype)
    af = attn_feat.reshape(N, 1)
    wr1x = wr1[:H].astype(cdtype)
    wr1a = wr1[H:H + 1]
    b1r = b1.reshape(1, DFF)
    b2r = b2.reshape(1, H)
    br1r = br1.reshape(1, GH)
    br2r = br2.reshape(1, 1)

    bm = min(512, N)
    bk = min(2048, DFF)
    grid = (N // bm, DFF // bk)

    out = pl.pallas_call(
        _ffn_body,
        grid=grid,
        in_specs=[
            pl.BlockSpec((bm, H), lambda i, k: (i, 0)),
            pl.BlockSpec((bm, 1), lambda i, k: (i, 0)),
            pl.BlockSpec((H, bk), lambda i, k: (0, k)),
            pl.BlockSpec((1, bk), lambda i, k: (0, k)),
            pl.BlockSpec((bk, H), lambda i, k: (k, 0)),
            pl.BlockSpec((1, H), lambda i, k: (0, 0)),
            pl.BlockSpec((H, GH), lambda i, k: (0, 0)),
            pl.BlockSpec((1, GH), lambda i, k: (0, 0)),
            pl.BlockSpec((1, GH), lambda i, k: (0, 0)),
            pl.BlockSpec((GH, 1), lambda i, k: (0, 0)),
            pl.BlockSpec((1, 1), lambda i, k: (0, 0)),
        ],
        out_specs=pl.BlockSpec((bm, H), lambda i, k: (i, 0)),
        out_shape=jax.ShapeDtypeStruct((N, H), jnp.float32),
        compiler_params=pltpu.CompilerParams(
            dimension_semantics=("parallel", "arbitrary")),
    )(xr, af, w1.astype(cdtype), b1r, w2.astype(cdtype), b2r,
      wr1x, wr1a, br1r, wr2, br2r)

    return out.reshape(B, S, H)
